# baseline (speedup 1.0000x reference)
import functools

import jax
import jax.numpy as jnp
from jax import lax
from jax.experimental import pallas as pl
from jax.experimental.pallas import tpu as pltpu
from jax.experimental.pallas import tpu_sc as plsc

_N = 10000
_D = 128
_NPAD = 10112
_NR_DEG = 10240
_NC = 2
_NS = 16
_NW = _NC * _NS
_K = 128
_ROWS_PER_TILE = _NPAD // _NS
_IDXB = 48
_F0 = 0.25
_TCB = 400


def _pad_to(x, m):
    return ((x + m - 1) // m) * m


def _make_seg_kernel(dd, rounds):
    mesh = plsc.VectorSubcoreMesh(core_axis_name="c", subcore_axis_name="s",
                                  num_cores=_NC, num_subcores=_NS)
    nrounds = len(rounds)
    out_t = tuple(jax.ShapeDtypeStruct((_NC, _NPAD, dd), jnp.float32)
                  for _ in range(nrounds))
    scratch = [
        pltpu.VMEM((_IDXB, _K), jnp.int32),
        pltpu.VMEM((_IDXB, _K), jnp.int32),
        pltpu.VMEM((_K, dd), jnp.float32),
        pltpu.VMEM((_K, dd), jnp.float32),
        pltpu.VMEM_SHARED((_NPAD, dd), jnp.float32),
        pltpu.SemaphoreType.DMA,
        pltpu.SemaphoreType.DMA,
    ]

    def body(zrow, *refs):
        tables = refs[0:3 * nrounds:3]
        srcs = refs[1:3 * nrounds:3]
        dsts = refs[2:3 * nrounds:3]
        outs = refs[3 * nrounds:4 * nrounds]
        src_v, dst_v, rows0, rows1, acc, sem0, sem1 = refs[4 * nrounds:]
        c = lax.axis_index("c")
        s = lax.axis_index("s")
        row0 = s * _ROWS_PER_TILE

        work = tuple((tables[r], srcs[r], dsts[r], rounds[r], outs[r])
                     for r in range(nrounds))
        for table, src_h, dst_h, (ca, cb), out_h in work:
            bufs = ((rows0, sem0), (rows1, sem1))

            def start(i, b):
                rows, sem = bufs[b]
                pltpu.async_copy(table.at[src_v.at[i]], rows, sem)

            def finish(i, b):
                rows, sem = bufs[b]
                pltpu.make_async_copy(table.at[src_v.at[i]], rows, sem).wait()
                pltpu.sync_copy(rows, acc.at[dst_v.at[i]], add=True)

            def run_stage(cbase, n):
                pltpu.sync_copy(src_h.at[pl.ds(cbase, n)],
                                src_v.at[pl.ds(0, n)])
                pltpu.sync_copy(dst_h.at[pl.ds(cbase, n)],
                                dst_v.at[pl.ds(0, n)])
                start(0, 0)
                if n > 1:
                    start(1, 1)

                def step(t, _):
                    i0 = 2 * t
                    finish(i0, 0)

                    @pl.when(i0 + 2 < n)
                    def _():
                        start(i0 + 2, 0)

                    finish(i0 + 1, 1)

                    @pl.when(i0 + 3 < n)
                    def _():
                        start(i0 + 3, 1)
                    return 0

                lax.fori_loop(0, n // 2, step, 0)
                if n % 2:
                    finish(n - 1, 0)

            pltpu.sync_copy(zrow, rows0)
            for r in range(_ROWS_PER_TILE // _K):
                pltpu.sync_copy(rows0, acc.at[pl.ds(row0 + r * _K, _K)])
            rem = _ROWS_PER_TILE % _K
            if rem:
                pltpu.sync_copy(
                    rows0.at[pl.ds(0, rem)],
                    acc.at[pl.ds(row0 + _ROWS_PER_TILE - rem, rem)])
            plsc.subcore_barrier()

            @pl.when(c == 0)
            def _():
                for off in range(0, ca, _IDXB):
                    n = min(_IDXB, ca - off)
                    run_stage(s * ca + off, n)

            @pl.when(c == 1)
            def _():
                for off in range(0, cb, _IDXB):
                    n = min(_IDXB, cb - off)
                    run_stage(_NS * ca + s * cb + off, n)

            plsc.subcore_barrier()
            pltpu.sync_copy(acc.at[pl.ds(row0, _ROWS_PER_TILE)],
                            out_h.at[c, pl.ds(row0, _ROWS_PER_TILE)])

    return pl.kernel(body, out_type=out_t, mesh=mesh, scratch_types=scratch)


def _make_deg_kernel(chunks_p, chunks_n):
    mesh = plsc.VectorSubcoreMesh(core_axis_name="c", subcore_axis_name="s",
                                  num_cores=_NC, num_subcores=_NS)
    nr = _NR_DEG // 128
    rpt = 8
    nact = nr // rpt
    out_t = (jax.ShapeDtypeStruct((_NC, nr, 128), jnp.float32),
             jax.ShapeDtypeStruct((_NC, nr, 128), jnp.float32))
    cmax = max(chunks_p, chunks_n)
    scratch = [
        pltpu.VMEM((cmax, _K), jnp.int32),
        pltpu.VMEM((nr, 128), jnp.float32),
        pltpu.VMEM((nr, 128), jnp.float32),
        pltpu.VMEM((nr,), jnp.int32),
        pltpu.VMEM_SHARED((nr, 128), jnp.float32),
        pltpu.SemaphoreType.DMA,
    ]

    def body(zeros_hbm, iota_hbm, dst_p, dst_n, out_p, out_n,
             dst_v, hist, zbuf, iota_v, acc, sem):
        c = lax.axis_index("c")
        s = lax.axis_index("s")
        row0 = s * rpt
        wid = c * _NS + s
        pltpu.sync_copy(zeros_hbm, zbuf)
        pltpu.sync_copy(iota_hbm, iota_v)
        ones16 = jnp.ones((16,), jnp.float32)

        for dst_h, nchunks, out_h in ((dst_p, chunks_p, out_p),
                                      (dst_n, chunks_n, out_n)):
            pltpu.sync_copy(zeros_hbm, hist)
            pltpu.sync_copy(dst_h.at[pl.ds(wid * nchunks, nchunks)],
                            dst_v.at[pl.ds(0, nchunks)])

            @pl.when(s < nact)
            def _():
                pltpu.sync_copy(zbuf.at[pl.ds(row0, rpt)],
                                acc.at[pl.ds(row0, rpt)])
            plsc.subcore_barrier()

            def chunk(i, _):
                for j in range(_K // 16):
                    d = dst_v[i, pl.ds(j * 16, 16)]
                    plsc.addupdate_scatter(
                        hist,
                        [lax.shift_right_logical(d, 7), lax.bitwise_and(d, 127)],
                        ones16)
                return 0

            lax.fori_loop(0, nchunks, chunk, 0)
            pltpu.sync_copy(hist, acc.at[iota_v], add=True)
            plsc.subcore_barrier()

            @pl.when(s < nact)
            def _():
                pltpu.sync_copy(acc.at[pl.ds(row0, rpt)],
                                out_h.at[c, pl.ds(row0, rpt)])
            plsc.subcore_barrier()

    return pl.kernel(
        body, out_type=out_t, mesh=mesh, scratch_types=scratch,
        compiler_params=pltpu.CompilerParams(needs_layout_passes=False))


def _tc1_body(x_ref, sp_ref, sn_ref, ivp_ref, ivn_ref, w_ref, outa_ref, outb_ref):
    z = (x_ref[...]
         + (sp_ref[0] + sp_ref[1]) * ivp_ref[...]
         - (sn_ref[0] + sn_ref[1]) * ivn_ref[...])
    h = jnp.dot(z, w_ref[...], preferred_element_type=jnp.float32)
    h = jnp.maximum(h, 0.0)
    outa_ref[...] = h[:, :128]
    outb_ref[...] = h[:, 128:]


def _tc2_body(xa_ref, xb_ref, spa_ref, sna_ref, spb_ref, snb_ref,
              ivp_ref, ivn_ref, w_ref, out_ref):
    ivp = ivp_ref[...]
    ivn = ivn_ref[...]
    za = xa_ref[...] + (spa_ref[0] + spa_ref[1]) * ivp - (sna_ref[0] + sna_ref[1]) * ivn
    zb = xb_ref[...] + (spb_ref[0] + spb_ref[1]) * ivp - (snb_ref[0] + snb_ref[1]) * ivn
    h = (jnp.dot(za, w_ref[0], preferred_element_type=jnp.float32)
         + jnp.dot(zb, w_ref[1], preferred_element_type=jnp.float32))
    h = jnp.maximum(h, 0.0)
    out_ref[...] = 0.25 * (h[:, :64] + h[:, 64:128] + h[:, 128:192] + h[:, 192:])


def _tc1(x, s1p, s1n, ivp, ivn, w1c):
    grid = (_N // _TCB,)
    part = lambda i: (0, i, 0)
    row = lambda i: (i, 0)
    return pl.pallas_call(
        _tc1_body,
        grid=grid,
        in_specs=[
            pl.BlockSpec((_TCB, _D), row),
            pl.BlockSpec((_NC, _TCB, _D), part),
            pl.BlockSpec((_NC, _TCB, _D), part),
            pl.BlockSpec((_TCB, 1), row),
            pl.BlockSpec((_TCB, 1), row),
            pl.BlockSpec((_D, 256), lambda i: (0, 0)),
        ],
        out_specs=[pl.BlockSpec((_TCB, _D), row),
                   pl.BlockSpec((_TCB, _D), row)],
        out_shape=[jax.ShapeDtypeStruct((_NPAD, _D), jnp.float32),
                   jax.ShapeDtypeStruct((_NPAD, _D), jnp.float32)],
    )(x, s1p, s1n, ivp, ivn, w1c)


def _tc2(xa, xb, s2ap, s2an, s2bp, s2bn, ivp, ivn, w2c):
    grid = (_N // _TCB,)
    part = lambda i: (0, i, 0)
    row = lambda i: (i, 0)
    return pl.pallas_call(
        _tc2_body,
        grid=grid,
        in_specs=[
            pl.BlockSpec((_TCB, _D), row),
            pl.BlockSpec((_TCB, _D), row),
            pl.BlockSpec((_NC, _TCB, _D), part),
            pl.BlockSpec((_NC, _TCB, _D), part),
            pl.BlockSpec((_NC, _TCB, _D), part),
            pl.BlockSpec((_NC, _TCB, _D), part),
            pl.BlockSpec((_TCB, 1), row),
            pl.BlockSpec((_TCB, 1), row),
            pl.BlockSpec((2, _D, 256), lambda i: (0, 0, 0)),
        ],
        out_specs=pl.BlockSpec((_TCB, 64), row),
        out_shape=jax.ShapeDtypeStruct((_N, 64), jnp.float32),
    )(xa, xb, s2ap, s2an, s2bp, s2bn, ivp, ivn, w2c)


def kernel(node_reps, adj_pos, adj_neg, W1, W2):
    e_pos = adj_pos.shape[1]
    e_neg = adj_neg.shape[1]
    rows_p = _pad_to(-(-e_pos // _K), 256)
    rows_n = _pad_to(-(-e_neg // _K), 256)
    epp = rows_p * _K
    epn = rows_n * _K

    def _split(rows):
        per_tile = rows // _NS
        ca = min(per_tile - 8, max(8, int(round(per_tile * _F0 / 8)) * 8))
        return ca, per_tile - ca

    ca_p, cb_p = _split(rows_p)
    ca_n, cb_n = _split(rows_n)

    sp = jnp.concatenate([adj_pos[0], jnp.zeros((epp - e_pos,), jnp.int32)]
                         ).reshape(-1, _K)
    dp = jnp.concatenate([adj_pos[1], jnp.full((epp - e_pos,), _N, jnp.int32)]
                         ).reshape(-1, _K)
    sn = jnp.concatenate([adj_neg[0], jnp.zeros((epn - e_neg,), jnp.int32)]
                         ).reshape(-1, _K)
    dn = jnp.concatenate([adj_neg[1], jnp.full((epn - e_neg,), _N, jnp.int32)]
                         ).reshape(-1, _K)

    nr = _NR_DEG // 128
    deg_k = _make_deg_kernel(rows_p // _NW, rows_n // _NW)
    zdeg = jnp.zeros((nr, 128), jnp.float32)
    iota = jnp.arange(nr, dtype=jnp.int32)
    deg_p_parts, deg_n_parts = deg_k(zdeg, iota, dp, dn)
    deg_p = (deg_p_parts[0] + deg_p_parts[1]).reshape(_NR_DEG)[:_N]
    deg_n = (deg_n_parts[0] + deg_n_parts[1]).reshape(_NR_DEG)[:_N]
    ivp = (1.0 / jnp.maximum(deg_p, 1.0)).reshape(_N, 1)
    ivn = (1.0 / jnp.maximum(deg_n, 1.0)).reshape(_N, 1)

    z128 = jnp.zeros((_K, _D), jnp.float32)

    seg2 = _make_seg_kernel(_D, [(ca_p, cb_p), (ca_n, cb_n)])
    s1p, s1n = seg2(z128, node_reps, sp, dp, node_reps, sn, dn)
    w1c = jnp.transpose(W1, (1, 0, 2)).reshape(_D, 256)
    x1a, x1b = _tc1(node_reps, s1p, s1n, ivp, ivn, w1c)

    s2ap, s2an = seg2(z128, x1a, sp, dp, x1a, sn, dn)
    s2bp, s2bn = seg2(z128, x1b, sp, dp, x1b, sn, dn)
    w2c = jnp.transpose(W2, (1, 0, 2)).reshape(256, 256).reshape(2, _D, 256)
    return _tc2(x1a, x1b, s2ap, s2an, s2bp, s2bn, ivp, ivn, w2c)


# speedup vs baseline: 1.0620x; 1.0620x over previous
import functools

import jax
import jax.numpy as jnp
from jax import lax
from jax.experimental import pallas as pl
from jax.experimental.pallas import tpu as pltpu
from jax.experimental.pallas import tpu_sc as plsc

_N = 10000
_D = 128
_NPAD = 10112
_NR_DEG = 10240
_NC = 2
_NS = 16
_NW = _NC * _NS
_K = 128
_ROWS_PER_TILE = _NPAD // _NS
_IDXB = 48
_F0 = 0.75
_TCB = 400


def _pad_to(x, m):
    return ((x + m - 1) // m) * m


def _make_seg_kernel(dd, rounds):
    mesh = plsc.VectorSubcoreMesh(core_axis_name="c", subcore_axis_name="s",
                                  num_cores=_NC, num_subcores=_NS)
    nrounds = len(rounds)
    out_t = tuple(jax.ShapeDtypeStruct((_NC, _NPAD, dd), jnp.float32)
                  for _ in range(nrounds))
    scratch = [
        pltpu.VMEM((_IDXB, _K), jnp.int32),
        pltpu.VMEM((_IDXB, _K), jnp.int32),
        pltpu.VMEM((_K, dd), jnp.float32),
        pltpu.VMEM((_K, dd), jnp.float32),
        pltpu.VMEM_SHARED((_NPAD, dd), jnp.float32),
        pltpu.SemaphoreType.DMA,
        pltpu.SemaphoreType.DMA,
    ]

    def body(zrow, *refs):
        tables = refs[0:3 * nrounds:3]
        srcs = refs[1:3 * nrounds:3]
        dsts = refs[2:3 * nrounds:3]
        outs = refs[3 * nrounds:4 * nrounds]
        src_v, dst_v, rows0, rows1, acc, sem0, sem1 = refs[4 * nrounds:]
        c = lax.axis_index("c")
        s = lax.axis_index("s")
        row0 = s * _ROWS_PER_TILE

        work = tuple((tables[r], srcs[r], dsts[r], rounds[r], outs[r])
                     for r in range(nrounds))
        for table, src_h, dst_h, (ca, cb), out_h in work:
            bufs = ((rows0, sem0), (rows1, sem1))

            def start(i, b):
                rows, sem = bufs[b]
                pltpu.async_copy(table.at[src_v.at[i]], rows, sem)

            def finish(i, b):
                rows, sem = bufs[b]
                pltpu.make_async_copy(table.at[src_v.at[i]], rows, sem).wait()
                pltpu.sync_copy(rows, acc.at[dst_v.at[i]], add=True)

            def run_stage(cbase, n):
                pltpu.sync_copy(src_h.at[pl.ds(cbase, n)],
                                src_v.at[pl.ds(0, n)])
                pltpu.sync_copy(dst_h.at[pl.ds(cbase, n)],
                                dst_v.at[pl.ds(0, n)])
                start(0, 0)
                if n > 1:
                    start(1, 1)

                def step(t, _):
                    i0 = 2 * t
                    finish(i0, 0)

                    @pl.when(i0 + 2 < n)
                    def _():
                        start(i0 + 2, 0)

                    finish(i0 + 1, 1)

                    @pl.when(i0 + 3 < n)
                    def _():
                        start(i0 + 3, 1)
                    return 0

                lax.fori_loop(0, n // 2, step, 0)
                if n % 2:
                    finish(n - 1, 0)

            pltpu.sync_copy(zrow, rows0)
            for r in range(_ROWS_PER_TILE // _K):
                pltpu.sync_copy(rows0, acc.at[pl.ds(row0 + r * _K, _K)])
            rem = _ROWS_PER_TILE % _K
            if rem:
                pltpu.sync_copy(
                    rows0.at[pl.ds(0, rem)],
                    acc.at[pl.ds(row0 + _ROWS_PER_TILE - rem, rem)])
            plsc.subcore_barrier()

            @pl.when(c == 0)
            def _():
                for off in range(0, ca, _IDXB):
                    n = min(_IDXB, ca - off)
                    run_stage(s * ca + off, n)

            @pl.when(c == 1)
            def _():
                for off in range(0, cb, _IDXB):
                    n = min(_IDXB, cb - off)
                    run_stage(_NS * ca + s * cb + off, n)

            plsc.subcore_barrier()
            pltpu.sync_copy(acc.at[pl.ds(row0, _ROWS_PER_TILE)],
                            out_h.at[c, pl.ds(row0, _ROWS_PER_TILE)])

    return pl.kernel(body, out_type=out_t, mesh=mesh, scratch_types=scratch)


def _make_deg_kernel(chunks_p, chunks_n):
    mesh = plsc.VectorSubcoreMesh(core_axis_name="c", subcore_axis_name="s",
                                  num_cores=_NC, num_subcores=_NS)
    nr = _NR_DEG // 128
    rpt = 8
    nact = nr // rpt
    out_t = (jax.ShapeDtypeStruct((_NC, nr, 128), jnp.float32),
             jax.ShapeDtypeStruct((_NC, nr, 128), jnp.float32))
    cmax = max(chunks_p, chunks_n)
    scratch = [
        pltpu.VMEM((cmax, _K), jnp.int32),
        pltpu.VMEM((nr, 128), jnp.float32),
        pltpu.VMEM((nr, 128), jnp.float32),
        pltpu.VMEM((nr,), jnp.int32),
        pltpu.VMEM_SHARED((nr, 128), jnp.float32),
        pltpu.SemaphoreType.DMA,
    ]

    def body(zeros_hbm, iota_hbm, dst_p, dst_n, out_p, out_n,
             dst_v, hist, zbuf, iota_v, acc, sem):
        c = lax.axis_index("c")
        s = lax.axis_index("s")
        row0 = s * rpt
        wid = c * _NS + s
        pltpu.sync_copy(zeros_hbm, zbuf)
        pltpu.sync_copy(iota_hbm, iota_v)
        ones16 = jnp.ones((16,), jnp.float32)

        for dst_h, nchunks, out_h in ((dst_p, chunks_p, out_p),
                                      (dst_n, chunks_n, out_n)):
            pltpu.sync_copy(zeros_hbm, hist)
            pltpu.sync_copy(dst_h.at[pl.ds(wid * nchunks, nchunks)],
                            dst_v.at[pl.ds(0, nchunks)])

            @pl.when(s < nact)
            def _():
                pltpu.sync_copy(zbuf.at[pl.ds(row0, rpt)],
                                acc.at[pl.ds(row0, rpt)])
            plsc.subcore_barrier()

            def chunk(i, _):
                for j in range(_K // 16):
                    d = dst_v[i, pl.ds(j * 16, 16)]
                    plsc.addupdate_scatter(
                        hist,
                        [lax.shift_right_logical(d, 7), lax.bitwise_and(d, 127)],
                        ones16)
                return 0

            lax.fori_loop(0, nchunks, chunk, 0)
            pltpu.sync_copy(hist, acc.at[iota_v], add=True)
            plsc.subcore_barrier()

            @pl.when(s < nact)
            def _():
                pltpu.sync_copy(acc.at[pl.ds(row0, rpt)],
                                out_h.at[c, pl.ds(row0, rpt)])
            plsc.subcore_barrier()

    return pl.kernel(
        body, out_type=out_t, mesh=mesh, scratch_types=scratch,
        compiler_params=pltpu.CompilerParams(needs_layout_passes=False))


def _tc1_body(x_ref, sp_ref, sn_ref, ivp_ref, ivn_ref, w_ref, outa_ref, outb_ref):
    z = (x_ref[...]
         + (sp_ref[0] + sp_ref[1]) * ivp_ref[...]
         - (sn_ref[0] + sn_ref[1]) * ivn_ref[...])
    h = jnp.dot(z, w_ref[...], preferred_element_type=jnp.float32)
    h = jnp.maximum(h, 0.0)
    outa_ref[...] = h[:, :128]
    outb_ref[...] = h[:, 128:]


def _tc2_body(xa_ref, xb_ref, spa_ref, sna_ref, spb_ref, snb_ref,
              ivp_ref, ivn_ref, w_ref, out_ref):
    ivp = ivp_ref[...]
    ivn = ivn_ref[...]
    za = xa_ref[...] + (spa_ref[0] + spa_ref[1]) * ivp - (sna_ref[0] + sna_ref[1]) * ivn
    zb = xb_ref[...] + (spb_ref[0] + spb_ref[1]) * ivp - (snb_ref[0] + snb_ref[1]) * ivn
    h = (jnp.dot(za, w_ref[0], preferred_element_type=jnp.float32)
         + jnp.dot(zb, w_ref[1], preferred_element_type=jnp.float32))
    h = jnp.maximum(h, 0.0)
    out_ref[...] = 0.25 * (h[:, :64] + h[:, 64:128] + h[:, 128:192] + h[:, 192:])


def _tc1(x, s1p, s1n, ivp, ivn, w1c):
    grid = (_N // _TCB,)
    part = lambda i: (0, i, 0)
    row = lambda i: (i, 0)
    return pl.pallas_call(
        _tc1_body,
        grid=grid,
        in_specs=[
            pl.BlockSpec((_TCB, _D), row),
            pl.BlockSpec((_NC, _TCB, _D), part),
            pl.BlockSpec((_NC, _TCB, _D), part),
            pl.BlockSpec((_TCB, 1), row),
            pl.BlockSpec((_TCB, 1), row),
            pl.BlockSpec((_D, 256), lambda i: (0, 0)),
        ],
        out_specs=[pl.BlockSpec((_TCB, _D), row),
                   pl.BlockSpec((_TCB, _D), row)],
        out_shape=[jax.ShapeDtypeStruct((_NPAD, _D), jnp.float32),
                   jax.ShapeDtypeStruct((_NPAD, _D), jnp.float32)],
    )(x, s1p, s1n, ivp, ivn, w1c)


def _tc2(xa, xb, s2ap, s2an, s2bp, s2bn, ivp, ivn, w2c):
    grid = (_N // _TCB,)
    part = lambda i: (0, i, 0)
    row = lambda i: (i, 0)
    return pl.pallas_call(
        _tc2_body,
        grid=grid,
        in_specs=[
            pl.BlockSpec((_TCB, _D), row),
            pl.BlockSpec((_TCB, _D), row),
            pl.BlockSpec((_NC, _TCB, _D), part),
            pl.BlockSpec((_NC, _TCB, _D), part),
            pl.BlockSpec((_NC, _TCB, _D), part),
            pl.BlockSpec((_NC, _TCB, _D), part),
            pl.BlockSpec((_TCB, 1), row),
            pl.BlockSpec((_TCB, 1), row),
            pl.BlockSpec((2, _D, 256), lambda i: (0, 0, 0)),
        ],
        out_specs=pl.BlockSpec((_TCB, 64), row),
        out_shape=jax.ShapeDtypeStruct((_N, 64), jnp.float32),
    )(xa, xb, s2ap, s2an, s2bp, s2bn, ivp, ivn, w2c)


def kernel(node_reps, adj_pos, adj_neg, W1, W2):
    e_pos = adj_pos.shape[1]
    e_neg = adj_neg.shape[1]
    rows_p = _pad_to(-(-e_pos // _K), 256)
    rows_n = _pad_to(-(-e_neg // _K), 256)
    epp = rows_p * _K
    epn = rows_n * _K

    def _split(rows):
        per_tile = rows // _NS
        ca = min(per_tile - 8, max(8, int(round(per_tile * _F0 / 8)) * 8))
        return ca, per_tile - ca

    ca_p, cb_p = _split(rows_p)
    ca_n, cb_n = _split(rows_n)

    sp = jnp.concatenate([adj_pos[0], jnp.zeros((epp - e_pos,), jnp.int32)]
                         ).reshape(-1, _K)
    dp = jnp.concatenate([adj_pos[1], jnp.full((epp - e_pos,), _N, jnp.int32)]
                         ).reshape(-1, _K)
    sn = jnp.concatenate([adj_neg[0], jnp.zeros((epn - e_neg,), jnp.int32)]
                         ).reshape(-1, _K)
    dn = jnp.concatenate([adj_neg[1], jnp.full((epn - e_neg,), _N, jnp.int32)]
                         ).reshape(-1, _K)

    nr = _NR_DEG // 128
    deg_k = _make_deg_kernel(rows_p // _NW, rows_n // _NW)
    zdeg = jnp.zeros((nr, 128), jnp.float32)
    iota = jnp.arange(nr, dtype=jnp.int32)
    deg_p_parts, deg_n_parts = deg_k(zdeg, iota, dp, dn)
    deg_p = (deg_p_parts[0] + deg_p_parts[1]).reshape(_NR_DEG)[:_N]
    deg_n = (deg_n_parts[0] + deg_n_parts[1]).reshape(_NR_DEG)[:_N]
    ivp = (1.0 / jnp.maximum(deg_p, 1.0)).reshape(_N, 1)
    ivn = (1.0 / jnp.maximum(deg_n, 1.0)).reshape(_N, 1)

    z128 = jnp.zeros((_K, _D), jnp.float32)

    seg2 = _make_seg_kernel(_D, [(ca_p, cb_p), (ca_n, cb_n)])
    s1p, s1n = seg2(z128, node_reps, sp, dp, node_reps, sn, dn)
    w1c = jnp.transpose(W1, (1, 0, 2)).reshape(_D, 256)
    x1a, x1b = _tc1(node_reps, s1p, s1n, ivp, ivn, w1c)

    s2ap, s2an = seg2(z128, x1a, sp, dp, x1a, sn, dn)
    s2bp, s2bn = seg2(z128, x1b, sp, dp, x1b, sn, dn)
    w2c = jnp.transpose(W2, (1, 0, 2)).reshape(256, 256).reshape(2, _D, 256)
    return _tc2(x1a, x1b, s2ap, s2an, s2bp, s2bn, ivp, ivn, w2c)
